# Initial kernel scaffold; baseline (speedup 1.0000x reference)
import jax
import jax.numpy as jnp
from jax.experimental import pallas as pl


def kernel(edge_feats, segment_ids, W, b):
    raise NotImplementedError("write your pallas kernel here")


# speedup vs baseline: 3.8168x; 3.8168x over previous
import jax
import jax.numpy as jnp
from jax.experimental import pallas as pl
from jax.experimental.pallas import tpu as pltpu

E = 320000
D = 128
S = 256
BE = 2000
NB = E // BE


def _fused_body(seg_ref, x_ref, w_ref, b_ref, out_ref, wout_ref):
    i = pl.program_id(0)
    x = x_ref[...]
    wvec = w_ref[...]
    logits = jax.lax.dot_general(
        x, wvec, (((1,), (0,)), ((), ())),
        preferred_element_type=jnp.float32)
    w = jax.nn.sigmoid(logits + b_ref[0, 0])
    wout_ref[...] = w

    seg = seg_ref[0, 0, :]
    sid = jax.lax.broadcasted_iota(jnp.int32, (BE, S), 1)
    onehot = (seg[:, None] == sid).astype(jnp.float32) * w
    partial = jax.lax.dot_general(
        onehot.astype(jnp.bfloat16), x.astype(jnp.bfloat16),
        (((0,), (0,)), ((), ())),
        preferred_element_type=jnp.float32)

    @pl.when(i == 0)
    def _():
        out_ref[...] = jnp.zeros_like(out_ref)

    out_ref[...] += partial


def kernel(edge_feats, segment_ids, W, b):
    seg3 = segment_ids.astype(jnp.int32).reshape(NB, 1, BE)
    b2 = b.astype(jnp.float32).reshape(1, 1)
    h, weights = pl.pallas_call(
        _fused_body,
        grid=(NB,),
        in_specs=[
            pl.BlockSpec((1, 1, BE), lambda i: (i, 0, 0)),
            pl.BlockSpec((BE, D), lambda i: (i, 0)),
            pl.BlockSpec((D, 1), lambda i: (0, 0)),
            pl.BlockSpec((1, 1), lambda i: (0, 0)),
        ],
        out_specs=[
            pl.BlockSpec((S, D), lambda i: (0, 0)),
            pl.BlockSpec((BE, 1), lambda i: (i, 0)),
        ],
        out_shape=[
            jax.ShapeDtypeStruct((S, D), jnp.float32),
            jax.ShapeDtypeStruct((E, 1), jnp.float32),
        ],
        compiler_params=pltpu.CompilerParams(
            dimension_semantics=("arbitrary",),
        ),
    )(seg3, edge_feats, W, b2)
    return (h, weights)


# speedup vs baseline: 7.4229x; 1.9448x over previous
import jax
import jax.numpy as jnp
from jax.experimental import pallas as pl
from jax.experimental.pallas import tpu as pltpu

E = 320000
D = 128
S = 256
BE = 2000
NB = E // BE


def _fused_body(seg_ref, x_ref, w_ref, b_ref, iota_ref, out_ref, wout_ref):
    i = pl.program_id(0)
    x = x_ref[...]
    xb = x.astype(jnp.bfloat16)
    w1 = w_ref[...].astype(jnp.bfloat16)
    logits_t = jax.lax.dot_general(
        w1, xb, (((1,), (1,)), ((), ())),
        preferred_element_type=jnp.float32)
    w_t = jax.nn.sigmoid(logits_t + b_ref[0, 0])
    wout_ref[...] = w_t.reshape(1, 1, BE)

    seg = seg_ref[0]
    onehot_t = jnp.where(iota_ref[...] == seg,
                         w_t.astype(jnp.bfloat16),
                         jnp.bfloat16(0))
    partial = jax.lax.dot_general(
        onehot_t, xb, (((1,), (0,)), ((), ())),
        preferred_element_type=jnp.float32)

    @pl.when(i == 0)
    def _():
        out_ref[...] = jnp.zeros_like(out_ref)

    out_ref[...] += partial


def kernel(edge_feats, segment_ids, W, b):
    seg3 = segment_ids.astype(jnp.bfloat16).reshape(NB, 1, BE)
    b2 = b.astype(jnp.float32).reshape(1, 1)
    w1 = W.astype(jnp.float32).reshape(1, D)
    iota = jnp.arange(S, dtype=jnp.bfloat16).reshape(S, 1)
    h, weights = pl.pallas_call(
        _fused_body,
        grid=(NB,),
        in_specs=[
            pl.BlockSpec((1, 1, BE), lambda i: (i, 0, 0)),
            pl.BlockSpec((BE, D), lambda i: (i, 0)),
            pl.BlockSpec((1, D), lambda i: (0, 0)),
            pl.BlockSpec((1, 1), lambda i: (0, 0)),
            pl.BlockSpec((S, 1), lambda i: (0, 0)),
        ],
        out_specs=[
            pl.BlockSpec((S, D), lambda i: (0, 0)),
            pl.BlockSpec((1, 1, BE), lambda i: (i, 0, 0)),
        ],
        out_shape=[
            jax.ShapeDtypeStruct((S, D), jnp.float32),
            jax.ShapeDtypeStruct((NB, 1, BE), jnp.float32),
        ],
        compiler_params=pltpu.CompilerParams(
            dimension_semantics=("arbitrary",),
        ),
    )(seg3, edge_feats, w1, b2, iota)
    return (h, weights.reshape(E, 1))


# speedup vs baseline: 13.6574x; 1.8399x over previous
import jax
import jax.numpy as jnp
from jax.experimental import pallas as pl
from jax.experimental.pallas import tpu as pltpu

E = 320000
D = 128
S = 256
BE = 8000
NB = E // BE
WS = 32


def _fused_body(meta_ref, seg_ref, x_ref, w_ref, b_ref, iota_ref,
                out_ref, wout_ref):
    i = pl.program_id(0)
    x = x_ref[...]
    xb = x.astype(jnp.bfloat16)
    w1 = w_ref[...].astype(jnp.bfloat16)
    logits_t = jax.lax.dot_general(
        w1, xb, (((1,), (1,)), ((), ())),
        preferred_element_type=jnp.float32)
    w_t = jax.nn.sigmoid(logits_t + b_ref[0, 0])
    wout_ref[...] = w_t.reshape(1, 1, BE)

    @pl.when(i == 0)
    def _():
        out_ref[...] = jnp.zeros_like(out_ref)

    seg = seg_ref[0]
    w_tb = w_t.astype(jnp.bfloat16)
    first = meta_ref[0, 0, 0]
    last = meta_ref[0, 0, 1]
    for jw in range(S // WS):
        base = jw * WS

        @pl.when((first < base + WS) & (last >= base))
        def _():
            onehot_w = jnp.where(iota_ref[base:base + WS, :] == seg,
                                 w_tb, jnp.bfloat16(0))
            out_ref[base:base + WS, :] += jax.lax.dot_general(
                onehot_w, xb, (((1,), (0,)), ((), ())),
                preferred_element_type=jnp.float32)


def kernel(edge_feats, segment_ids, W, b):
    segi = segment_ids.astype(jnp.int32)
    meta = jnp.stack([segi[::BE], segi[BE - 1::BE]], axis=1).reshape(NB, 1, 2)
    seg3 = segi.astype(jnp.bfloat16).reshape(NB, 1, BE)
    b2 = b.astype(jnp.float32).reshape(1, 1)
    w1 = W.astype(jnp.float32).reshape(1, D)
    iota = jnp.arange(S, dtype=jnp.bfloat16).reshape(S, 1)
    h, weights = pl.pallas_call(
        _fused_body,
        grid=(NB,),
        in_specs=[
            pl.BlockSpec((1, 1, 2), lambda i: (i, 0, 0),
                         memory_space=pltpu.SMEM),
            pl.BlockSpec((1, 1, BE), lambda i: (i, 0, 0)),
            pl.BlockSpec((BE, D), lambda i: (i, 0)),
            pl.BlockSpec((1, D), lambda i: (0, 0)),
            pl.BlockSpec((1, 1), lambda i: (0, 0)),
            pl.BlockSpec((S, 1), lambda i: (0, 0)),
        ],
        out_specs=[
            pl.BlockSpec((S, D), lambda i: (0, 0)),
            pl.BlockSpec((1, 1, BE), lambda i: (i, 0, 0)),
        ],
        out_shape=[
            jax.ShapeDtypeStruct((S, D), jnp.float32),
            jax.ShapeDtypeStruct((NB, 1, BE), jnp.float32),
        ],
        compiler_params=pltpu.CompilerParams(
            dimension_semantics=("arbitrary",),
        ),
    )(meta, seg3, edge_feats, w1, b2, iota)
    return (h, weights.reshape(E, 1))
